# Initial kernel scaffold; baseline (speedup 1.0000x reference)
import jax
import jax.numpy as jnp
from jax.experimental import pallas as pl


def kernel(x, Wlin, att_src, att_dst, bias, ln_w, ln_b):
    raise NotImplementedError("write your pallas kernel here")


# speedup vs baseline: 833.1146x; 833.1146x over previous
import jax
import jax.numpy as jnp
from jax.experimental import pallas as pl
from jax.experimental.pallas import tpu as pltpu

H = 224
W = 224
N = H * W
C = 96
HEADS = 8
D = 12
TN = 3584
P = 256
RB = TN // P
NB = N // P
TPB = N // TN
TNE = TN + 2 * P

OFFSETS = [(dr, dc) for dr in (-1, 0, 1) for dc in (-1, 0, 1)]
NEG = -1e30


def _gat_grid_kernel(xl_ref, xc_ref, xr_ref, wt_ref, asrc_ref, adst_ref,
                     bias_ref, lnw_ref, lnb_ref, out_ref):
    i = pl.program_id(1)
    j0 = i * TN

    x_ext = jnp.concatenate([xl_ref[0], xc_ref[0], xr_ref[0]], axis=1)
    xw_ext = jnp.dot(wt_ref[:], x_ext, preferred_element_type=jnp.float32)
    s_ext = jnp.dot(asrc_ref[:], xw_ext, preferred_element_type=jnp.float32)
    t = jnp.dot(adst_ref[:], xw_ext[:, P:P + TN],
                preferred_element_type=jnp.float32)

    idx = jax.lax.broadcasted_iota(jnp.int32, (1, TN), 1) + j0
    r = idx // W
    c = idx % W

    a_list = []
    m = jnp.full((HEADS, TN), NEG, jnp.float32)
    for dr, dc in OFFSETS:
        delta = dr * W + dc
        sk = s_ext[:, P - delta:P - delta + TN]
        z = sk + t
        a = jnp.where(z >= 0, z, 0.2 * z)
        if dr != 0 or dc != 0:
            mask = None
            if dr != 0:
                rs = r - dr
                mask = (rs >= 0) & (rs < H)
            if dc != 0:
                cs = c - dc
                mc = (cs >= 0) & (cs < W)
                mask = mc if mask is None else (mask & mc)
            a = jnp.where(mask, a, NEG)
        a_list.append(a)
        m = jnp.maximum(m, a)

    den = jnp.zeros((HEADS, TN), jnp.float32)
    e_list = []
    for (dr, dc), a in zip(OFFSETS, a_list):
        e = jnp.exp(a - m)
        if dr == 0 and dc == 0:
            e = e * 2.0
        e_list.append(e)
        den = den + e
    inv = 1.0 / (den + 1e-16)

    rep = (jax.lax.broadcasted_iota(jnp.int32, (C, HEADS), 0) // D ==
           jax.lax.broadcasted_iota(jnp.int32, (C, HEADS), 1)
           ).astype(jnp.float32)

    acc = jnp.zeros((C, TN), jnp.float32)
    for (dr, dc), e in zip(OFFSETS, e_list):
        delta = dr * W + dc
        w_full = jnp.dot(rep, e * inv, preferred_element_type=jnp.float32)
        acc = acc + w_full * xw_ext[:, P - delta:P - delta + TN]

    o = acc + bias_ref[:]
    o = jnp.where(o > 0, o, jnp.exp(jnp.minimum(o, 0.0)) - 1.0)
    mu = jnp.mean(o, axis=0, keepdims=True)
    var = jnp.mean((o - mu) ** 2, axis=0, keepdims=True)
    o = (o - mu) * jax.lax.rsqrt(var + 1e-5) * lnw_ref[:] + lnb_ref[:]
    out_ref[0] = o


def kernel(x, Wlin, att_src, att_dst, bias, ln_w, ln_b):
    B = x.shape[0]
    x3 = x.reshape(B, C, N)
    wt = Wlin.T
    eye = jnp.eye(HEADS, dtype=jnp.float32)
    a_src = (att_src[:, None, :] * eye[:, :, None]).reshape(HEADS, C)
    a_dst = (att_dst[:, None, :] * eye[:, :, None]).reshape(HEADS, C)

    out = pl.pallas_call(
        _gat_grid_kernel,
        grid=(B, TPB),
        in_specs=[
            pl.BlockSpec((1, C, P),
                         lambda b, i: (b, 0, jnp.maximum(i * RB - 1, 0))),
            pl.BlockSpec((1, C, TN), lambda b, i: (b, 0, i)),
            pl.BlockSpec((1, C, P),
                         lambda b, i: (b, 0, jnp.minimum(i * RB + RB, NB - 1))),
            pl.BlockSpec((C, C), lambda b, i: (0, 0)),
            pl.BlockSpec((HEADS, C), lambda b, i: (0, 0)),
            pl.BlockSpec((HEADS, C), lambda b, i: (0, 0)),
            pl.BlockSpec((C, 1), lambda b, i: (0, 0)),
            pl.BlockSpec((C, 1), lambda b, i: (0, 0)),
            pl.BlockSpec((C, 1), lambda b, i: (0, 0)),
        ],
        out_specs=pl.BlockSpec((1, C, TN), lambda b, i: (b, 0, i)),
        out_shape=jax.ShapeDtypeStruct((B, C, N), jnp.float32),
        compiler_params=pltpu.CompilerParams(
            dimension_semantics=("parallel", "parallel")),
    )(x3, x3, x3, wt, a_src, a_dst,
      bias.reshape(C, 1), ln_w.reshape(C, 1), ln_b.reshape(C, 1))
    return out.reshape(B, C, H, W)
